# baseline (speedup 1.0000x reference)
import functools

import jax
import jax.numpy as jnp
from jax import lax
from jax.experimental import pallas as pl
from jax.experimental.pallas import tpu as pltpu
from jax.experimental.pallas import tpu_sc as plsc

N_NODES_K = 100000
N_PAD = 100096
N_EDGES_K = 6400000
CH = 3200
CH_DEG = 12800
NC = 2
NS = 16
NW = NC * NS
PER_TILE = N_PAD // NS
L = 16


def _fill_const(ref, n, value):
    v = jnp.full((L,), value, dtype=jnp.float32)

    def body(i, _):
        ref[pl.ds(i * L, L)] = v
        return 0

    lax.fori_loop(0, n // L, body, 0)


def _edge_pass_body(with_gather, ch, *refs):
    if with_gather:
        (y_hbm, edge_hbm, out_hbm, y_tile, src_b0, src_b1, dst_b0, dst_b1,
         val_b0, val_b1, sem0, sem1, in_sem, acc) = refs
        src_b = (src_b0, src_b1)
        val_b = (val_b0, val_b1)
    else:
        (edge_hbm, out_hbm, dst_b0, dst_b1, val_b0, sem0, sem1,
         acc) = refs
        val_b = (val_b0, val_b0)
    dst_b = (dst_b0, dst_b1)
    sems = (sem0, sem1)

    cid = lax.axis_index("c")
    sid = lax.axis_index("s")
    wid = sid * NC + cid

    if with_gather:
        tbl = pltpu.async_copy(y_hbm, y_tile, in_sem)

    _fill_const(val_b0, CH, 0.0)
    pltpu.sync_copy(val_b0.at[pl.ds(0, CH)],
                    acc.at[pl.ds(sid * PER_TILE, CH)])
    pltpu.sync_copy(val_b0.at[pl.ds(0, PER_TILE - CH)],
                    acc.at[pl.ds(sid * PER_TILE + CH, PER_TILE - CH)])

    if with_gather:
        tbl.wait()
    else:
        _fill_const(val_b0, ch, 1.0)

    plsc.subcore_barrier()

    def scatter_wait(pb):
        pltpu.make_async_copy(val_b[pb], acc.at[dst_b[pb]], sems[pb]).wait()

    def process_chunk(k, pb):
        base = (wid + k * NW) * ch
        if with_gather:
            d_dst = pltpu.async_copy(edge_hbm.at[1, pl.ds(base, ch)],
                                     dst_b[pb], in_sem)
            d_src = pltpu.async_copy(edge_hbm.at[0, pl.ds(base, ch)],
                                     src_b[pb], in_sem)
            d_dst.wait()
            d_src.wait()

            @plsc.parallel_loop(0, ch // L, unroll=8)
            def _(j):
                idx = src_b[pb][pl.ds(j * L, L)]
                val_b[pb][pl.ds(j * L, L)] = plsc.load_gather(y_tile, [idx])
        else:
            pltpu.sync_copy(edge_hbm.at[1, pl.ds(base, ch)], dst_b[pb])
        pltpu.async_copy(val_b[pb], acc.at[dst_b[pb]], sems[pb], add=True)

    n_chunks = N_EDGES_K // ch
    nchunks_w = (n_chunks - wid + NW - 1) // NW
    npair = nchunks_w // 2

    def pair_body(m, _):
        for pb in range(2):
            @pl.when(m > 0)
            def _():
                scatter_wait(pb)

            process_chunk(2 * m + pb, pb)
        return 0

    lax.fori_loop(0, npair, pair_body, 0)

    @pl.when(nchunks_w % 2 == 1)
    def _():
        scatter_wait(0)
        process_chunk(2 * npair, 0)

    scatter_wait(0)
    scatter_wait(1)

    plsc.subcore_barrier()
    obase = cid * N_PAD + sid * PER_TILE
    pltpu.sync_copy(acc.at[pl.ds(sid * PER_TILE, CH)],
                    val_b0.at[pl.ds(0, CH)])
    pltpu.sync_copy(val_b0.at[pl.ds(0, CH)], out_hbm.at[pl.ds(obase, CH)])
    rem = PER_TILE - CH
    pltpu.sync_copy(acc.at[pl.ds(sid * PER_TILE + CH, rem)],
                    val_b0.at[pl.ds(0, rem)])
    pltpu.sync_copy(val_b0.at[pl.ds(0, rem)],
                    out_hbm.at[pl.ds(obase + CH, rem)])


_SC_MESH = plsc.VectorSubcoreMesh(core_axis_name="c", subcore_axis_name="s")
_SC_PARAMS = pltpu.CompilerParams(needs_layout_passes=False)

_edge_gather_pass = pl.kernel(
    functools.partial(_edge_pass_body, True, CH),
    out_type=jax.ShapeDtypeStruct((NC * N_PAD,), jnp.float32),
    mesh=_SC_MESH,
    compiler_params=_SC_PARAMS,
    scratch_types=[
        pltpu.VMEM((N_PAD,), jnp.float32),
        pltpu.VMEM((CH,), jnp.int32),
        pltpu.VMEM((CH,), jnp.int32),
        pltpu.VMEM((CH,), jnp.int32),
        pltpu.VMEM((CH,), jnp.int32),
        pltpu.VMEM((CH,), jnp.float32),
        pltpu.VMEM((CH,), jnp.float32),
        pltpu.SemaphoreType.DMA,
        pltpu.SemaphoreType.DMA,
        pltpu.SemaphoreType.DMA,
        pltpu.VMEM_SHARED((N_PAD,), jnp.float32),
    ],
)

_deg_pass = pl.kernel(
    functools.partial(_edge_pass_body, False, CH_DEG),
    out_type=jax.ShapeDtypeStruct((NC * N_PAD,), jnp.float32),
    mesh=_SC_MESH,
    compiler_params=_SC_PARAMS,
    scratch_types=[
        pltpu.VMEM((CH_DEG,), jnp.int32),
        pltpu.VMEM((CH_DEG,), jnp.int32),
        pltpu.VMEM((CH_DEG,), jnp.float32),
        pltpu.SemaphoreType.DMA,
        pltpu.SemaphoreType.DMA,
        pltpu.VMEM_SHARED((N_PAD,), jnp.float32),
    ],
)


def _stage1_body(degp_ref, xp_ref, d_ref, y_ref):
    deg = degp_ref[0] + degp_ref[1] + 1.0
    dd = lax.rsqrt(deg)
    d_ref[...] = dd
    y_ref[...] = dd * xp_ref[...]


def _stage2_body(sp_ref, y_ref, d_ref, w1_ref, b1_ref, w2_ref, g_ref):
    dd = d_ref[...]
    s = dd * (sp_ref[0] + sp_ref[1] + y_ref[...])
    acc = jnp.zeros_like(s)
    for j in range(35):
        acc = acc + jax.nn.sigmoid(s * w1_ref[j] + b1_ref[j]) * w2_ref[j]
    g_ref[...] = dd * acc


def _stage3_body(up_ref, g_ref, d_ref, b2_ref, o_ref):
    u = d_ref[...] * (up_ref[0] + up_ref[1] + g_ref[...]) + b2_ref[0]
    o_ref[...] = jax.nn.sigmoid(u)


_V = pl.BlockSpec(memory_space=pltpu.VMEM)
_S = pl.BlockSpec(memory_space=pltpu.SMEM)
_R2 = (N_PAD // 128, 128)

_stage1 = pl.pallas_call(
    _stage1_body,
    in_specs=[_V, _V],
    out_specs=(_V, _V),
    out_shape=(jax.ShapeDtypeStruct(_R2, jnp.float32),
               jax.ShapeDtypeStruct(_R2, jnp.float32)),
)

_stage2 = pl.pallas_call(
    _stage2_body,
    in_specs=[_V, _V, _V, _S, _S, _S],
    out_specs=_V,
    out_shape=jax.ShapeDtypeStruct(_R2, jnp.float32),
)

_stage3 = pl.pallas_call(
    _stage3_body,
    in_specs=[_V, _V, _V, _S],
    out_specs=_V,
    out_shape=jax.ShapeDtypeStruct(_R2, jnp.float32),
)


def kernel(x, edge_index, edge_attr, W1, b1, W2, b2):
    del edge_attr
    xp = jnp.pad(x[:, 0], (0, N_PAD - N_NODES_K))
    e3 = edge_index

    degp = _deg_pass(e3)
    dd, y = _stage1(degp.reshape(NC, *_R2), xp.reshape(_R2))

    sp = _edge_gather_pass(y.reshape(N_PAD), e3)
    g = _stage2(sp.reshape(NC, *_R2), y, dd,
                W1.reshape(35), b1, W2.reshape(35))

    up = _edge_gather_pass(g.reshape(N_PAD), e3)
    o = _stage3(up.reshape(NC, *_R2), g, dd, b2)

    return o.reshape(N_PAD)[:N_NODES_K].reshape(N_NODES_K, 1)


# speedup vs baseline: 1.0762x; 1.0762x over previous
import functools

import jax
import jax.numpy as jnp
from jax import lax
from jax.experimental import pallas as pl
from jax.experimental.pallas import tpu as pltpu
from jax.experimental.pallas import tpu_sc as plsc

N_NODES_K = 100000
N_PAD = 100096
N_EDGES_K = 6400000
CH = 2560
NB = 3
NC = 2
NS = 16
NW = NC * NS
PER_TILE = N_PAD // NS
L = 16
ZC = 2560


def _fill_const(ref, n, value):
    v = jnp.full((L,), value, dtype=jnp.float32)

    def body(i, _):
        ref[pl.ds(i * L, L)] = v
        return 0

    lax.fori_loop(0, n // L, body, 0)


def _edge_pass_body(with_gather, *refs):
    if with_gather:
        (y_hbm, edge_hbm, out_hbm, y_tile,
         src_b0, src_b1, src_b2, dst_b0, dst_b1, dst_b2,
         val_b0, val_b1, val_b2,
         sc_sem0, sc_sem1, sc_sem2, in_sem0, in_sem1, in_sem2, acc) = refs
        src_b = (src_b0, src_b1, src_b2)
        val_b = (val_b0, val_b1, val_b2)
    else:
        (edge_hbm, out_hbm, dst_b0, dst_b1, dst_b2, val_b0,
         sc_sem0, sc_sem1, sc_sem2, in_sem0, in_sem1, in_sem2, acc) = refs
        val_b = (val_b0, val_b0, val_b0)
    dst_b = (dst_b0, dst_b1, dst_b2)
    sc_sems = (sc_sem0, sc_sem1, sc_sem2)
    in_sems = (in_sem0, in_sem1, in_sem2)

    cid = lax.axis_index("c")
    sid = lax.axis_index("s")
    wid = sid * NC + cid

    if with_gather:
        tbl = pltpu.async_copy(y_hbm, y_tile, in_sems[0])

    _fill_const(val_b0, ZC, 0.0)
    for off in range(0, PER_TILE, ZC):
        pc = min(ZC, PER_TILE - off)
        pltpu.sync_copy(val_b0.at[pl.ds(0, pc)],
                        acc.at[pl.ds(sid * PER_TILE + off, pc)])

    if with_gather:
        tbl.wait()
    else:
        _fill_const(val_b0, CH, 1.0)

    n_chunks = N_EDGES_K // CH
    nw = (n_chunks - wid + NW - 1) // NW

    def in_descs(k, pb):
        base = (wid + k * NW) * CH
        d_dst = pltpu.make_async_copy(edge_hbm.at[1, pl.ds(base, CH)],
                                      dst_b[pb], in_sems[pb])
        if with_gather:
            d_src = pltpu.make_async_copy(edge_hbm.at[0, pl.ds(base, CH)],
                                          src_b[pb], in_sems[pb])
            return (d_dst, d_src)
        return (d_dst,)

    def sc_wait(pb):
        pltpu.make_async_copy(val_b[pb], acc.at[dst_b[pb]],
                              sc_sems[pb]).wait()

    def prefetch(k, pb):
        @pl.when(k < nw)
        def _():
            @pl.when(k >= NB)
            def _():
                sc_wait(pb)
            for d in in_descs(k, pb):
                d.start()

    def body(j, pb):
        @pl.when(j < nw)
        def _():
            for d in in_descs(j, pb):
                d.wait()
            prefetch(j + 1, (pb + 1) % NB)
            if with_gather:
                @plsc.parallel_loop(0, CH // L, unroll=8)
                def _(jj):
                    idx = src_b[pb][pl.ds(jj * L, L)]
                    val_b[pb][pl.ds(jj * L, L)] = plsc.load_gather(
                        y_tile, [idx])
            pltpu.async_copy(val_b[pb], acc.at[dst_b[pb]], sc_sems[pb],
                             add=True)

    plsc.subcore_barrier()
    prefetch(0, 0)

    def tri_body(m, _):
        for pb in range(NB):
            body(NB * m + pb, pb)
        return 0

    lax.fori_loop(0, nw // NB, tri_body, 0)
    jt = (nw // NB) * NB
    body(jt, 0)
    body(jt + 1, 1)

    sc_wait(0)
    sc_wait(1)
    sc_wait(2)

    plsc.subcore_barrier()
    obase = cid * N_PAD + sid * PER_TILE
    for off in range(0, PER_TILE, ZC):
        pc = min(ZC, PER_TILE - off)
        pltpu.sync_copy(acc.at[pl.ds(sid * PER_TILE + off, pc)],
                        val_b0.at[pl.ds(0, pc)])
        pltpu.sync_copy(val_b0.at[pl.ds(0, pc)],
                        out_hbm.at[pl.ds(obase + off, pc)])


_SC_MESH = plsc.VectorSubcoreMesh(core_axis_name="c", subcore_axis_name="s")
_SC_PARAMS = pltpu.CompilerParams(needs_layout_passes=False)

_edge_gather_pass = pl.kernel(
    functools.partial(_edge_pass_body, True),
    out_type=jax.ShapeDtypeStruct((NC * N_PAD,), jnp.float32),
    mesh=_SC_MESH,
    compiler_params=_SC_PARAMS,
    scratch_types=(
        [pltpu.VMEM((N_PAD,), jnp.float32)]
        + [pltpu.VMEM((CH,), jnp.int32) for _ in range(NB)]
        + [pltpu.VMEM((CH,), jnp.int32) for _ in range(NB)]
        + [pltpu.VMEM((CH,), jnp.float32) for _ in range(NB)]
        + [pltpu.SemaphoreType.DMA for _ in range(2 * NB)]
        + [pltpu.VMEM_SHARED((N_PAD,), jnp.float32)]
    ),
)

_deg_pass = pl.kernel(
    functools.partial(_edge_pass_body, False),
    out_type=jax.ShapeDtypeStruct((NC * N_PAD,), jnp.float32),
    mesh=_SC_MESH,
    compiler_params=_SC_PARAMS,
    scratch_types=(
        [pltpu.VMEM((CH,), jnp.int32) for _ in range(NB)]
        + [pltpu.VMEM((CH,), jnp.float32)]
        + [pltpu.SemaphoreType.DMA for _ in range(2 * NB)]
        + [pltpu.VMEM_SHARED((N_PAD,), jnp.float32)]
    ),
)


def _stage1_body(degp_ref, xp_ref, d_ref, y_ref):
    deg = degp_ref[0] + degp_ref[1] + 1.0
    dd = lax.rsqrt(deg)
    d_ref[...] = dd
    y_ref[...] = dd * xp_ref[...]


def _stage2_body(sp_ref, y_ref, d_ref, w1_ref, b1_ref, w2_ref, g_ref):
    dd = d_ref[...]
    s = dd * (sp_ref[0] + sp_ref[1] + y_ref[...])
    acc = jnp.zeros_like(s)
    for j in range(35):
        acc = acc + jax.nn.sigmoid(s * w1_ref[j] + b1_ref[j]) * w2_ref[j]
    g_ref[...] = dd * acc


def _stage3_body(up_ref, g_ref, d_ref, b2_ref, o_ref):
    u = d_ref[...] * (up_ref[0] + up_ref[1] + g_ref[...]) + b2_ref[0]
    o_ref[...] = jax.nn.sigmoid(u)


_V = pl.BlockSpec(memory_space=pltpu.VMEM)
_S = pl.BlockSpec(memory_space=pltpu.SMEM)
_R2 = (N_PAD // 128, 128)

_stage1 = pl.pallas_call(
    _stage1_body,
    in_specs=[_V, _V],
    out_specs=(_V, _V),
    out_shape=(jax.ShapeDtypeStruct(_R2, jnp.float32),
               jax.ShapeDtypeStruct(_R2, jnp.float32)),
)

_stage2 = pl.pallas_call(
    _stage2_body,
    in_specs=[_V, _V, _V, _S, _S, _S],
    out_specs=_V,
    out_shape=jax.ShapeDtypeStruct(_R2, jnp.float32),
)

_stage3 = pl.pallas_call(
    _stage3_body,
    in_specs=[_V, _V, _V, _S],
    out_specs=_V,
    out_shape=jax.ShapeDtypeStruct(_R2, jnp.float32),
)


def kernel(x, edge_index, edge_attr, W1, b1, W2, b2):
    del edge_attr
    xp = jnp.pad(x[:, 0], (0, N_PAD - N_NODES_K))
    e3 = edge_index

    degp = _deg_pass(e3)
    dd, y = _stage1(degp.reshape(NC, *_R2), xp.reshape(_R2))

    sp = _edge_gather_pass(y.reshape(N_PAD), e3)
    g = _stage2(sp.reshape(NC, *_R2), y, dd,
                W1.reshape(35), b1, W2.reshape(35))

    up = _edge_gather_pass(g.reshape(N_PAD), e3)
    o = _stage3(up.reshape(NC, *_R2), g, dd, b2)

    return o.reshape(N_PAD)[:N_NODES_K].reshape(N_NODES_K, 1)
